# baseline (speedup 1.0000x reference)
import functools

import jax
import jax.numpy as jnp
from jax import lax
from jax.experimental import pallas as pl
from jax.experimental.pallas import tpu as pltpu, tpu_sc as plsc

_R = 8
_K = 2 * _R + 1

_NC = 2
_NS = 16
_NW = _NC * _NS
_BT = 5


@functools.lru_cache(maxsize=None)
def _build_fill(H: int, W: int):
    NBW = 4
    NBH = _NW // NBW
    BH = H // NBH
    BW = W // NBW
    nbt = BW // _BT
    nch = W // 16
    MW = (BW - 1 + W + 15) // 16 * 16

    mesh = plsc.VectorSubcoreMesh(
        core_axis_name="c", subcore_axis_name="s",
        num_cores=_NC, num_subcores=_NS)

    @functools.partial(
        pl.kernel,
        out_type=jax.ShapeDtypeStruct((1, 1, H, W, H, W), jnp.float32),
        mesh=mesh,
        scratch_types=[
            pltpu.VMEM((_K, _K), jnp.float32),
            pltpu.VMEM((_K, MW), jnp.float32),
            [pltpu.VMEM((_BT, H, W), jnp.float32)] * 2,
            [pltpu.SemaphoreType.DMA] * 2,
        ],
    )
    def fill(biases_hbm, out_hbm, bv, master, tbs, sems):
        pltpu.sync_copy(biases_hbm, bv)

        wid = lax.axis_index("s") * _NC + lax.axis_index("c")
        bh = wid // NBW
        bw = wid - bh * NBW
        h1base = bh * BH
        w1base = bw * BW
        cmin = (W - 1) - (w1base + BW - 1)

        zeros16 = jnp.zeros((16,), jnp.float32)

        def zrow(r, carry):
            for tb in tbs:
                for i in range(_BT):
                    for j in range(nch):
                        tb[i, r, pl.ds(j * 16, 16)] = zeros16
            return carry

        lax.fori_loop(0, H, zrow, 0)

        cc = cmin - (W - 1 - _R)
        for kh in range(_K):
            blo = bv[kh, pl.ds(0, 16)]
            bhi = bv[kh, pl.ds(1, 16)]
            for j in range(MW // 16):
                lo = 16 * j + cc
                hit = (lo <= _K - 1) & (lo >= -15)

                @pl.when(hit)
                def _sel(j=j, lo=lo, blo=blo, bhi=bhi, kh=kh):
                    idx = lax.iota(jnp.int32, 16) + lo
                    acc = jnp.zeros((16,), jnp.float32)
                    for kw in range(_K):
                        b = blo[kw] if kw < 16 else bhi[15]
                        acc = jnp.where(idx == kw, b, acc)
                    master[kh, pl.ds(j * 16, 16)] = acc

                @pl.when(jnp.logical_not(hit))
                def _zero(j=j, kh=kh):
                    master[kh, pl.ds(j * 16, 16)] = zeros16

        def build_batch(q, tb):
            qh = q // nbt
            h1 = h1base + qh
            bi = q - qh * nbt
            w1lo = w1base + _BT * bi

            @pl.when(h1 - (_R + 1) >= 0)
            def _zstale():
                for i in range(_BT):
                    for j in range(nch):
                        tb[i, h1 - (_R + 1), pl.ds(j * 16, 16)] = zeros16

            def krow(kh, carry):
                h2 = h1 - _R + kh

                @pl.when((h2 >= 0) & (h2 < H))
                def _row():
                    for i in range(_BT):
                        woff = (W - 1) - (w1lo + i) - cmin
                        for j in range(nch):
                            tb[i, h2, pl.ds(j * 16, 16)] = (
                                master[kh, pl.ds(woff + j * 16, 16)])

                return carry

            lax.fori_loop(0, _K, krow, 0)
            return h1, w1lo

        def dst(h1, w1lo):
            return out_hbm.at[0, 0, h1, pl.ds(w1lo, _BT)]

        nq = nbt * BH

        for b in range(2):
            h1a, w1a = build_batch(jnp.int32(b), tbs[b])
            pltpu.async_copy(tbs[b], dst(h1a, w1a), sems[b])

        def obody(t, carry):
            for b in range(2):
                q = 2 * t + b
                qh = q // nbt
                pltpu.make_async_copy(
                    tbs[b],
                    dst(h1base + qh, w1base + _BT * (q - qh * nbt)),
                    sems[b]).wait()
                h1, w1lo = build_batch(q, tbs[b])
                pltpu.async_copy(tbs[b], dst(h1, w1lo), sems[b])
            return carry

        lax.fori_loop(1, nq // 2, obody, 0)

        for b in range(2):
            q = nq - 2 + b
            qh = q // nbt
            pltpu.make_async_copy(
                tbs[b],
                dst(h1base + qh, w1base + _BT * (q - qh * nbt)),
                sems[b]).wait()

    return fill


def kernel(feat, biases, all_h1s, all_w1s, all_h2s, all_w2s):
    H, W = feat.shape[-2], feat.shape[-1]
    out = _build_fill(H, W)(biases.astype(jnp.float32))
    return out.astype(feat.dtype)


# speedup vs baseline: 1.0087x; 1.0087x over previous
import functools

import jax
import jax.numpy as jnp
from jax import lax
from jax.experimental import pallas as pl
from jax.experimental.pallas import tpu as pltpu, tpu_sc as plsc

_R = 8
_K = 2 * _R + 1

_NC = 2
_NS = 16
_NW = _NC * _NS
_BT = 4


@functools.lru_cache(maxsize=None)
def _build_fill(H: int, W: int):
    NBW = 4
    NBH = _NW // NBW
    BH = H // NBH
    BW = W // NBW
    nbt = BW // _BT
    nch = W // 16
    MW = (BW - 1 + W + 15) // 16 * 16

    mesh = plsc.VectorSubcoreMesh(
        core_axis_name="c", subcore_axis_name="s",
        num_cores=_NC, num_subcores=_NS)

    @functools.partial(
        pl.kernel,
        out_type=jax.ShapeDtypeStruct((1, 1, H, W, H, W), jnp.float32),
        mesh=mesh,
        scratch_types=[
            pltpu.VMEM((_K, _K), jnp.float32),
            pltpu.VMEM((_K, MW), jnp.float32),
            [pltpu.VMEM((_BT, H, W), jnp.float32)] * 2,
            [pltpu.SemaphoreType.DMA] * 2,
        ],
    )
    def fill(biases_hbm, out_hbm, bv, master, tbs, sems):
        pltpu.sync_copy(biases_hbm, bv)

        wid = lax.axis_index("s") * _NC + lax.axis_index("c")
        bh = wid // NBW
        bw = wid - bh * NBW
        h1base = bh * BH
        w1base = bw * BW
        cmin = (W - 1) - (w1base + BW - 1)

        zeros16 = jnp.zeros((16,), jnp.float32)

        def zrow(r, carry):
            for tb in tbs:
                for i in range(_BT):
                    for j in range(nch):
                        tb[i, r, pl.ds(j * 16, 16)] = zeros16
            return carry

        lax.fori_loop(0, H, zrow, 0)

        cc = cmin - (W - 1 - _R)
        for kh in range(_K):
            blo = bv[kh, pl.ds(0, 16)]
            bhi = bv[kh, pl.ds(1, 16)]
            for j in range(MW // 16):
                lo = 16 * j + cc
                hit = (lo <= _K - 1) & (lo >= -15)

                @pl.when(hit)
                def _sel(j=j, lo=lo, blo=blo, bhi=bhi, kh=kh):
                    idx = lax.iota(jnp.int32, 16) + lo
                    acc = jnp.zeros((16,), jnp.float32)
                    for kw in range(_K):
                        b = blo[kw] if kw < 16 else bhi[15]
                        acc = jnp.where(idx == kw, b, acc)
                    master[kh, pl.ds(j * 16, 16)] = acc

                @pl.when(jnp.logical_not(hit))
                def _zero(j=j, kh=kh):
                    master[kh, pl.ds(j * 16, 16)] = zeros16

        def build_batch(q, tb):
            qh = q // nbt
            h1 = h1base + qh
            bi = q - qh * nbt
            w1lo = w1base + _BT * bi

            @pl.when(h1 - (_R + 1) >= 0)
            def _zstale():
                for i in range(_BT):
                    for j in range(nch):
                        tb[i, h1 - (_R + 1), pl.ds(j * 16, 16)] = zeros16

            def krow(kh, carry):
                h2 = h1 - _R + kh

                @pl.when((h2 >= 0) & (h2 < H))
                def _row():
                    for i in range(_BT):
                        woff = (W - 1) - (w1lo + i) - cmin
                        for j in range(nch):
                            tb[i, h2, pl.ds(j * 16, 16)] = (
                                master[kh, pl.ds(woff + j * 16, 16)])

                return carry

            lax.fori_loop(0, _K, krow, 0)
            return h1, w1lo

        def dst(h1, w1lo):
            return out_hbm.at[0, 0, h1, pl.ds(w1lo, _BT)]

        nq = nbt * BH

        for b in range(2):
            h1a, w1a = build_batch(jnp.int32(b), tbs[b])
            pltpu.async_copy(tbs[b], dst(h1a, w1a), sems[b])

        def obody(t, carry):
            for b in range(2):
                q = 2 * t + b
                qh = q // nbt
                pltpu.make_async_copy(
                    tbs[b],
                    dst(h1base + qh, w1base + _BT * (q - qh * nbt)),
                    sems[b]).wait()
                h1, w1lo = build_batch(q, tbs[b])
                pltpu.async_copy(tbs[b], dst(h1, w1lo), sems[b])
            return carry

        lax.fori_loop(1, nq // 2, obody, 0)

        for b in range(2):
            q = nq - 2 + b
            qh = q // nbt
            pltpu.make_async_copy(
                tbs[b],
                dst(h1base + qh, w1base + _BT * (q - qh * nbt)),
                sems[b]).wait()

    return fill


def kernel(feat, biases, all_h1s, all_w1s, all_h2s, all_w2s):
    H, W = feat.shape[-2], feat.shape[-1]
    out = _build_fill(H, W)(biases.astype(jnp.float32))
    return out.astype(feat.dtype)
